# baseline (speedup 1.0000x reference)
import math

import jax
import jax.numpy as jnp
import numpy as np
from jax import lax
from jax.experimental import pallas as pl
from jax.experimental.pallas import tpu as pltpu
from jax.experimental.pallas import tpu_sc as plsc

_VOCAB = 1000000
_SIZE = 64
_MAX_SEQ_LEN = 1000
_BATCH = 4096
_SEQ = 200
_SCALE = np.float32(math.sqrt(_SIZE))

_NC = 2
_NS = 16
_NW = _NC * _NS

_SEQ_PER_W = _BATCH // _NW
_SEQ_PER_CHUNK = 2
_CHUNKS = _SEQ_PER_W // _SEQ_PER_CHUNK
_ROWS_PER_CHUNK = _SEQ_PER_CHUNK * _SEQ
_GATHER = 100
_NGATHER = _ROWS_PER_CHUNK // _GATHER
_IDX_ROWS = _SEQ_PER_W * _SEQ // _GATHER


def _make_pe(max_seq_len, size):
    pe = np.zeros((max_seq_len, size), dtype=np.float32)
    position = np.arange(0, max_seq_len, dtype=np.float32)[:, None]
    div_term = np.exp(
        np.arange(0, size, 2, dtype=np.float32) * -(math.log(10000.0) / size))
    pe[:, 0::2] = np.sin(position * div_term)
    pe[:, 1::2] = np.cos(position * div_term)
    return pe


_PE = _make_pe(_MAX_SEQ_LEN, _SIZE)[:_SEQ]


def _body(table_hbm, x_hbm, pe_hbm, out_hbm,
          idx_v, rows0, rows1, pe_v, gsem0, gsem1, ssem):
    wid = lax.axis_index("s") * _NC + lax.axis_index("c")
    rows = (rows0, rows1)
    gsem = (gsem0, gsem1)

    i0 = pl.multiple_of(wid * _IDX_ROWS, 8)
    pltpu.sync_copy(x_hbm.at[pl.ds(i0, _IDX_ROWS)], idx_v)
    pltpu.sync_copy(pe_hbm, pe_v)

    def fire_gathers(c, par):
        for g in range(_NGATHER):
            pltpu.async_copy(
                table_hbm.at[idx_v.at[c * _NGATHER + g]],
                rows[par].at[pl.ds(g * _GATHER, _GATHER)], gsem[par])

    def wait_gathers(par):
        pltpu.make_async_copy(
            table_hbm.at[pl.ds(0, _ROWS_PER_CHUNK)], rows[par],
            gsem[par]).wait()

    def store(c, par):
        row0 = pl.multiple_of((wid * _CHUNKS + c) * _ROWS_PER_CHUNK, 8)
        pltpu.async_copy(rows[par], out_hbm.at[pl.ds(row0, _ROWS_PER_CHUNK)],
                         ssem)

    def wait_store(par):
        pltpu.make_async_copy(
            rows[par], out_hbm.at[pl.ds(0, _ROWS_PER_CHUNK)], ssem).wait()

    def compute(par):
        buf = rows[par]

        @pl.loop(0, _SEQ)
        def _pos(l):
            pes = [pe_v[l, pl.ds(k * 16, 16)] for k in range(_SIZE // 16)]
            for s in range(_SEQ_PER_CHUNK):
                r = s * _SEQ + l
                for k in range(_SIZE // 16):
                    sl = pl.ds(k * 16, 16)
                    buf[r, sl] = buf[r, sl] * _SCALE + pes[k]

    fire_gathers(0, 0)

    @pl.loop(0, _CHUNKS, step=2)
    def _outer(t):
        @pl.when(t > 0)
        def _():
            wait_store(1)
        fire_gathers(t + 1, 1)
        wait_gathers(0)
        compute(0)
        store(t, 0)

        wait_store(0)

        @pl.when(t + 2 < _CHUNKS)
        def _():
            fire_gathers(t + 2, 0)
        wait_gathers(1)
        compute(1)
        store(t + 1, 1)

    wait_store(1)


def kernel(x, emb_table):
    b, seq = x.shape
    assert (b, seq) == (_BATCH, _SEQ) and emb_table.shape == (_VOCAB, _SIZE)
    x2d = x.astype(jnp.int32).reshape(b * seq // _GATHER, _GATHER)
    pe = jnp.asarray(_PE)

    run = pl.kernel(
        _body,
        out_type=jax.ShapeDtypeStruct((b * seq, _SIZE), jnp.float32),
        mesh=plsc.VectorSubcoreMesh(core_axis_name="c", subcore_axis_name="s"),
        compiler_params=pltpu.CompilerParams(
            use_tc_tiling_on_sc=False, skip_device_barrier=True),
        scratch_types=[
            pltpu.VMEM((_IDX_ROWS, _GATHER), jnp.int32),
            pltpu.VMEM((_ROWS_PER_CHUNK, _SIZE), jnp.float32),
            pltpu.VMEM((_ROWS_PER_CHUNK, _SIZE), jnp.float32),
            pltpu.VMEM((_SEQ, _SIZE), jnp.float32),
            pltpu.SemaphoreType.DMA,
            pltpu.SemaphoreType.DMA,
            pltpu.SemaphoreType.DMA,
        ],
    )
    out = run(emb_table, x2d, pe)
    return out.reshape(b, seq, _SIZE)
